# baseline (speedup 1.0000x reference)
import functools

import jax
import jax.numpy as jnp
from jax import lax
from jax.experimental import pallas as pl
from jax.experimental.pallas import tpu as pltpu
from jax.experimental.pallas import tpu_sc as plsc

B = 8
N = 4096
F = 64
E = 131072
IN_DIM = N * F
H = 128
L = 64

NC = 2
NS = 16
GPC = B // NC
EPT = E // NS
CH = 128
NCHUNK = EPT // CH
NBUF = 8
LOOK = 4
RPT = N // NS

_f32 = jnp.float32


def _sc_agg_body(x_hbm, src_hbm, dst_hbm, sums_hbm, cnt_hbm,
                 src_v, dst_v, rb0, rb1, rb2, rb3, rb4, rb5, rb6, rb7,
                 hist_v, zbuf, cacc_v, ctmp_v, acc_sh, cntp_sh,
                 gs0, gs1, gs2, gs3, gs4, gs5, gs6, gs7,
                 ss0, ss1, ss2, ss3, ss4, ss5, ss6, ss7):
    c = lax.axis_index("c")
    s = lax.axis_index("s")
    rbufs = (rb0, rb1, rb2, rb3, rb4, rb5, rb6, rb7)
    gsems = (gs0, gs1, gs2, gs3, gs4, gs5, gs6, gs7)
    ssems = (ss0, ss1, ss2, ss3, ss4, ss5, ss6, ss7)
    zeros16 = jnp.zeros((16,), _f32)
    ones16 = jnp.ones((16,), _f32)

    def _zb(i, _):
        zbuf[i // (F // 16), pl.ds((i % (F // 16)) * 16, 16)] = zeros16
        return 0
    lax.fori_loop(0, RPT * (F // 16), _zb, 0)

    def _round(r, _):
        g = c * GPC + r
        row = g * NS + s
        pltpu.sync_copy(zbuf, acc_sh.at[pl.ds(s * RPT, RPT)])

        def _zh(i, __):
            hist_v[pl.ds(i * 16, 16)] = zeros16
            return 0
        lax.fori_loop(0, N // 16, _zh, 0)

        pltpu.sync_copy(src_hbm.at[row], src_v)
        pltpu.sync_copy(dst_hbm.at[row], dst_v)
        plsc.subcore_barrier()

        for p in range(LOOK):
            pltpu.async_copy(x_hbm.at[src_v.at[p]], rbufs[p], gsems[p])

        def _wait_scatter(p, j):
            pltpu.make_async_copy(rbufs[p], acc_sh.at[dst_v.at[j]],
                                  ssems[p]).wait()

        def _outer(k, __):
            for p in range(NBUF):
                j = k * NBUF + p
                pltpu.make_async_copy(x_hbm.at[src_v.at[j]], rbufs[p],
                                      gsems[p]).wait()
                for v in range(CH // 16):
                    dvec = dst_v[j, pl.ds(v * 16, 16)]
                    plsc.addupdate_scatter(hist_v, [dvec], ones16)
                pltpu.async_copy(rbufs[p], acc_sh.at[dst_v.at[j]], ssems[p],
                                 add=True)
                j2 = j + LOOK
                p2 = (p + LOOK) % NBUF

                @pl.when(j2 >= NBUF)
                def _():
                    _wait_scatter(p2, j)

                @pl.when(j2 < NCHUNK)
                def _():
                    pltpu.async_copy(x_hbm.at[src_v.at[j2]], rbufs[p2],
                                     gsems[p2])
            return 0
        lax.fori_loop(0, NCHUNK // NBUF, _outer, 0)
        for p in range(LOOK, NBUF):
            _wait_scatter(p, p)

        pltpu.sync_copy(hist_v, cntp_sh.at[s])
        plsc.subcore_barrier()

        pltpu.sync_copy(acc_sh.at[pl.ds(s * RPT, RPT)],
                        sums_hbm.at[pl.ds(g * N + s * RPT, RPT)])
        pltpu.sync_copy(cntp_sh.at[0, pl.ds(s * RPT, RPT)], cacc_v)
        for t in range(1, NS):
            pltpu.sync_copy(cntp_sh.at[t, pl.ds(s * RPT, RPT)], ctmp_v)
            for q in range(RPT // 16):
                sl = pl.ds(q * 16, 16)
                cacc_v[sl] = cacc_v[sl] + ctmp_v[sl]
        pltpu.sync_copy(cacc_v, cnt_hbm.at[g * NS + s])
        plsc.subcore_barrier()
        return 0

    lax.fori_loop(0, GPC, _round, 0)


_sc_agg = functools.partial(
    pl.kernel,
    out_type=(jax.ShapeDtypeStruct((B * N, F), _f32),
              jax.ShapeDtypeStruct((B * NS, RPT), _f32)),
    mesh=plsc.VectorSubcoreMesh(core_axis_name="c", subcore_axis_name="s",
                                num_cores=NC, num_subcores=NS),
    compiler_params=pltpu.CompilerParams(needs_layout_passes=False,
                                         use_tc_tiling_on_sc=False),
    scratch_types=[
        pltpu.VMEM((NCHUNK, CH), jnp.int32),
        pltpu.VMEM((NCHUNK, CH), jnp.int32),
        *([pltpu.VMEM((CH, F), _f32)] * NBUF),
        pltpu.VMEM((N,), _f32),
        pltpu.VMEM((RPT, F), _f32),
        pltpu.VMEM((RPT,), _f32),
        pltpu.VMEM((RPT,), _f32),
        pltpu.VMEM_SHARED((N, F), _f32),
        pltpu.VMEM_SHARED((NS, N), _f32),
        *([pltpu.SemaphoreType.DMA] * (2 * NBUF)),
    ],
)(_sc_agg_body)


def _conv_body(sums_ref, cnt_ref, x_ref, wm_ref, bm_ref, out_ref):
    cnt = cnt_ref[...]
    g = sums_ref[...] / jnp.maximum(cnt, 1.0)
    m = jnp.dot(g, wm_ref[...], preferred_element_type=_f32)
    v = m + bm_ref[...] * (cnt > 0).astype(_f32) + x_ref[...]
    out_ref[...] = jnp.where(v >= 0, v, 0.01 * v)


_CONV_RB = 8192


def _conv(sums2d, cnt2d, x2d, W_msg, b_msg_row):
    rows = B * N
    return pl.pallas_call(
        _conv_body,
        grid=(rows // _CONV_RB,),
        in_specs=[
            pl.BlockSpec((_CONV_RB, F), lambda i: (i, 0)),
            pl.BlockSpec((_CONV_RB, 1), lambda i: (i, 0)),
            pl.BlockSpec((_CONV_RB, F), lambda i: (i, 0)),
            pl.BlockSpec((F, F), lambda i: (0, 0)),
            pl.BlockSpec((1, F), lambda i: (0, 0)),
        ],
        out_specs=pl.BlockSpec((_CONV_RB, F), lambda i: (i, 0)),
        out_shape=jax.ShapeDtypeStruct((rows, F), _f32),
    )(sums2d, cnt2d, x2d, W_msg, b_msg_row)


def _enc_body(xs_ref, w1_ref, b1_ref, out_ref, acc_ref):
    k = pl.program_id(0)

    @pl.when(k == 0)
    def _():
        acc_ref[...] = jnp.zeros_like(acc_ref)

    acc_ref[...] += jnp.dot(xs_ref[...], w1_ref[...],
                            preferred_element_type=_f32)

    @pl.when(k == pl.num_programs(0) - 1)
    def _():
        out_ref[...] = jnp.maximum(acc_ref[...] + b1_ref[...], 0.0)


_ENC_KB = 8192


def _enc(xs, enc_W1, b1_row):
    return pl.pallas_call(
        _enc_body,
        grid=(IN_DIM // _ENC_KB,),
        in_specs=[
            pl.BlockSpec((B, _ENC_KB), lambda k: (0, k)),
            pl.BlockSpec((_ENC_KB, H), lambda k: (k, 0)),
            pl.BlockSpec((1, H), lambda k: (0, 0)),
        ],
        out_specs=pl.BlockSpec((B, H), lambda k: (0, 0)),
        out_shape=jax.ShapeDtypeStruct((B, H), _f32),
        scratch_shapes=[pltpu.VMEM((B, H), _f32)],
    )(xs, enc_W1, b1_row)


def _latent_body(h1_ref, wm_ref, bm_ref, wl_ref, bl_ref, w1_ref, b1_ref,
                 eps_ref, mean_ref, lv_ref, d1_ref):
    h = h1_ref[...]
    mean = jnp.dot(h, wm_ref[...], preferred_element_type=_f32) + bm_ref[...]
    lv = jnp.dot(h, wl_ref[...], preferred_element_type=_f32) + bl_ref[...]
    z = mean + jnp.exp(0.5 * lv) * eps_ref[...]
    d1 = jnp.dot(z, w1_ref[...], preferred_element_type=_f32) + b1_ref[...]
    mean_ref[...] = mean
    lv_ref[...] = lv
    d1_ref[...] = jnp.maximum(d1, 0.0)


def _latent(h1, W_mean, bm_row, W_logvar, bl_row, dec_W1, db1_row, eps):
    return pl.pallas_call(
        _latent_body,
        out_shape=(jax.ShapeDtypeStruct((B, L), _f32),
                   jax.ShapeDtypeStruct((B, L), _f32),
                   jax.ShapeDtypeStruct((B, H), _f32)),
    )(h1, W_mean, bm_row, W_logvar, bl_row, dec_W1, db1_row, eps)


def _dec_body(d1_ref, w2_ref, b2_ref, out_ref):
    y = jnp.dot(d1_ref[...], w2_ref[...], preferred_element_type=_f32)
    out_ref[...] = jax.nn.sigmoid(y + b2_ref[...])


_DEC_CB = 8192


def _dec(d1, dec_W2, b2_row):
    return pl.pallas_call(
        _dec_body,
        grid=(IN_DIM // _DEC_CB,),
        in_specs=[
            pl.BlockSpec((B, H), lambda j: (0, 0)),
            pl.BlockSpec((H, _DEC_CB), lambda j: (0, j)),
            pl.BlockSpec((1, _DEC_CB), lambda j: (0, j)),
        ],
        out_specs=pl.BlockSpec((B, _DEC_CB), lambda j: (0, j)),
        out_shape=jax.ShapeDtypeStruct((B, IN_DIM), _f32),
    )(d1, dec_W2, b2_row)


def kernel(x, edge_index, W_msg, b_msg, enc_W1, enc_b1, W_mean, b_mean,
           W_logvar, b_logvar, dec_W1, dec_b1, dec_W2, dec_b2):
    x2d = x.reshape(B * N, F)
    src = edge_index[:, 0, :] + (jnp.arange(B, dtype=jnp.int32) * N)[:, None]
    dst = edge_index[:, 1, :]
    src_r = src.reshape(B * NS, NCHUNK, CH)
    dst_r = dst.reshape(B * NS, NCHUNK, CH)

    sums2d, cnt = _sc_agg(x2d, src_r, dst_r)
    cnt2d = cnt.reshape(B * N, 1)

    h2d = _conv(sums2d, cnt2d, x2d, W_msg, b_msg.reshape(1, F))
    xs = h2d.reshape(B, IN_DIM)
    h1 = _enc(xs, enc_W1, enc_b1.reshape(1, H))
    eps = jax.random.normal(jax.random.key(42), (B, L), dtype=_f32)
    mean, log_var, d1 = _latent(h1, W_mean, b_mean.reshape(1, L),
                                W_logvar, b_logvar.reshape(1, L),
                                dec_W1, dec_b1.reshape(1, H), eps)
    x_hat = _dec(d1, dec_W2, dec_b2.reshape(1, IN_DIM))
    return (x_hat, mean, log_var)


# speedup vs baseline: 1.1270x; 1.1270x over previous
import functools

import jax
import jax.numpy as jnp
from jax import lax
from jax.experimental import pallas as pl
from jax.experimental.pallas import tpu as pltpu
from jax.experimental.pallas import tpu_sc as plsc

B = 8
N = 4096
F = 64
E = 131072
IN_DIM = N * F
H = 128
L = 64

NC = 2
NS = 16
GPC = B // NC
EPT = E // NS
CH = 128
NCHUNK = EPT // CH
NBUF = 8
LOOK = 4
RPT = N // NS

_f32 = jnp.float32
_bf16 = jnp.bfloat16


def _sc_agg_body(x_hbm, src_hbm, dst_hbm, sums_hbm, cnt_hbm,
                 src_v, dst_v, rb0, rb1, rb2, rb3, rb4, rb5, rb6, rb7,
                 hist_v, zbuf, cacc_v, ctmp_v, acc_sh, cntp_sh,
                 gs0, gs1, gs2, gs3, gs4, gs5, gs6, gs7,
                 ss0, ss1, ss2, ss3, ss4, ss5, ss6, ss7):
    c = lax.axis_index("c")
    s = lax.axis_index("s")
    rbufs = (rb0, rb1, rb2, rb3, rb4, rb5, rb6, rb7)
    gsems = (gs0, gs1, gs2, gs3, gs4, gs5, gs6, gs7)
    ssems = (ss0, ss1, ss2, ss3, ss4, ss5, ss6, ss7)
    zeros16 = jnp.zeros((16,), _f32)
    ones16 = jnp.ones((16,), _f32)
    zeros32 = jnp.zeros((32,), _bf16)

    def _zb(i, _):
        zbuf[i // (F // 32), pl.ds((i % (F // 32)) * 32, 32)] = zeros32
        return 0
    lax.fori_loop(0, RPT * (F // 32), _zb, 0)

    def _round(r, _):
        g = c * GPC + r
        row = g * NS + s
        pltpu.sync_copy(zbuf, acc_sh.at[pl.ds(s * RPT, RPT)])

        def _zh(i, __):
            hist_v[pl.ds(i * 16, 16)] = zeros16
            return 0
        lax.fori_loop(0, N // 16, _zh, 0)

        pltpu.sync_copy(src_hbm.at[row], src_v)
        pltpu.sync_copy(dst_hbm.at[row], dst_v)
        plsc.subcore_barrier()

        for p in range(LOOK):
            pltpu.async_copy(x_hbm.at[src_v.at[p]], rbufs[p], gsems[p])

        def _wait_scatter(p, j):
            pltpu.make_async_copy(rbufs[p], acc_sh.at[dst_v.at[j]],
                                  ssems[p]).wait()

        def _outer(k, __):
            for p in range(NBUF):
                j = k * NBUF + p
                pltpu.make_async_copy(x_hbm.at[src_v.at[j]], rbufs[p],
                                      gsems[p]).wait()
                for v in range(CH // 16):
                    dvec = dst_v[j, pl.ds(v * 16, 16)]
                    plsc.addupdate_scatter(hist_v, [dvec], ones16)
                pltpu.async_copy(rbufs[p], acc_sh.at[dst_v.at[j]], ssems[p],
                                 add=True)
                j2 = j + LOOK
                p2 = (p + LOOK) % NBUF

                @pl.when(j2 >= NBUF)
                def _():
                    _wait_scatter(p2, j)

                @pl.when(j2 < NCHUNK)
                def _():
                    pltpu.async_copy(x_hbm.at[src_v.at[j2]], rbufs[p2],
                                     gsems[p2])
            return 0
        lax.fori_loop(0, NCHUNK // NBUF, _outer, 0)
        for p in range(LOOK, NBUF):
            _wait_scatter(p, p)

        pltpu.sync_copy(hist_v, cntp_sh.at[s])
        plsc.subcore_barrier()

        pltpu.sync_copy(acc_sh.at[pl.ds(s * RPT, RPT)],
                        sums_hbm.at[pl.ds(g * N + s * RPT, RPT)])
        pltpu.sync_copy(cntp_sh.at[0, pl.ds(s * RPT, RPT)], cacc_v)
        for t in range(1, NS):
            pltpu.sync_copy(cntp_sh.at[t, pl.ds(s * RPT, RPT)], ctmp_v)
            for q in range(RPT // 16):
                sl = pl.ds(q * 16, 16)
                cacc_v[sl] = cacc_v[sl] + ctmp_v[sl]
        pltpu.sync_copy(cacc_v, cnt_hbm.at[g * NS + s])
        plsc.subcore_barrier()
        return 0

    lax.fori_loop(0, GPC, _round, 0)


_sc_agg = functools.partial(
    pl.kernel,
    out_type=(jax.ShapeDtypeStruct((B * N, F), _bf16),
              jax.ShapeDtypeStruct((B * NS, RPT), _f32)),
    mesh=plsc.VectorSubcoreMesh(core_axis_name="c", subcore_axis_name="s",
                                num_cores=NC, num_subcores=NS),
    compiler_params=pltpu.CompilerParams(needs_layout_passes=False,
                                         use_tc_tiling_on_sc=False),
    scratch_types=[
        pltpu.VMEM((NCHUNK, CH), jnp.int32),
        pltpu.VMEM((NCHUNK, CH), jnp.int32),
        *([pltpu.VMEM((CH, F), _bf16)] * NBUF),
        pltpu.VMEM((N,), _f32),
        pltpu.VMEM((RPT, F), _bf16),
        pltpu.VMEM((RPT,), _f32),
        pltpu.VMEM((RPT,), _f32),
        pltpu.VMEM_SHARED((N, F), _bf16),
        pltpu.VMEM_SHARED((NS, N), _f32),
        *([pltpu.SemaphoreType.DMA] * (2 * NBUF)),
    ],
)(_sc_agg_body)


def _conv_body(sums_ref, cnt_ref, x_ref, wm_ref, bm_ref, out_ref):
    cnt = cnt_ref[...]
    g = sums_ref[...].astype(_f32) / jnp.maximum(cnt, 1.0)
    m = jnp.dot(g, wm_ref[...], preferred_element_type=_f32)
    v = m + bm_ref[...] * (cnt > 0).astype(_f32) + x_ref[...]
    out_ref[...] = jnp.where(v >= 0, v, 0.01 * v)


_CONV_RB = 8192


def _conv(sums2d, cnt2d, x2d, W_msg, b_msg_row):
    rows = B * N
    return pl.pallas_call(
        _conv_body,
        grid=(rows // _CONV_RB,),
        in_specs=[
            pl.BlockSpec((_CONV_RB, F), lambda i: (i, 0)),
            pl.BlockSpec((_CONV_RB, 1), lambda i: (i, 0)),
            pl.BlockSpec((_CONV_RB, F), lambda i: (i, 0)),
            pl.BlockSpec((F, F), lambda i: (0, 0)),
            pl.BlockSpec((1, F), lambda i: (0, 0)),
        ],
        out_specs=pl.BlockSpec((_CONV_RB, F), lambda i: (i, 0)),
        out_shape=jax.ShapeDtypeStruct((rows, F), _f32),
    )(sums2d, cnt2d, x2d, W_msg, b_msg_row)


def _enc_body(xs_ref, w1_ref, b1_ref, out_ref, acc_ref):
    k = pl.program_id(0)

    @pl.when(k == 0)
    def _():
        acc_ref[...] = jnp.zeros_like(acc_ref)

    acc_ref[...] += jnp.dot(xs_ref[...], w1_ref[...],
                            preferred_element_type=_f32)

    @pl.when(k == pl.num_programs(0) - 1)
    def _():
        out_ref[...] = jnp.maximum(acc_ref[...] + b1_ref[...], 0.0)


_ENC_KB = 8192


def _enc(xs, enc_W1, b1_row):
    return pl.pallas_call(
        _enc_body,
        grid=(IN_DIM // _ENC_KB,),
        in_specs=[
            pl.BlockSpec((B, _ENC_KB), lambda k: (0, k)),
            pl.BlockSpec((_ENC_KB, H), lambda k: (k, 0)),
            pl.BlockSpec((1, H), lambda k: (0, 0)),
        ],
        out_specs=pl.BlockSpec((B, H), lambda k: (0, 0)),
        out_shape=jax.ShapeDtypeStruct((B, H), _f32),
        scratch_shapes=[pltpu.VMEM((B, H), _f32)],
    )(xs, enc_W1, b1_row)


def _latent_body(h1_ref, wm_ref, bm_ref, wl_ref, bl_ref, w1_ref, b1_ref,
                 eps_ref, mean_ref, lv_ref, d1_ref):
    h = h1_ref[...]
    mean = jnp.dot(h, wm_ref[...], preferred_element_type=_f32) + bm_ref[...]
    lv = jnp.dot(h, wl_ref[...], preferred_element_type=_f32) + bl_ref[...]
    z = mean + jnp.exp(0.5 * lv) * eps_ref[...]
    d1 = jnp.dot(z, w1_ref[...], preferred_element_type=_f32) + b1_ref[...]
    mean_ref[...] = mean
    lv_ref[...] = lv
    d1_ref[...] = jnp.maximum(d1, 0.0)


def _latent(h1, W_mean, bm_row, W_logvar, bl_row, dec_W1, db1_row, eps):
    return pl.pallas_call(
        _latent_body,
        out_shape=(jax.ShapeDtypeStruct((B, L), _f32),
                   jax.ShapeDtypeStruct((B, L), _f32),
                   jax.ShapeDtypeStruct((B, H), _f32)),
    )(h1, W_mean, bm_row, W_logvar, bl_row, dec_W1, db1_row, eps)


def _dec_body(d1_ref, w2_ref, b2_ref, out_ref):
    y = jnp.dot(d1_ref[...], w2_ref[...], preferred_element_type=_f32)
    out_ref[...] = jax.nn.sigmoid(y + b2_ref[...])


_DEC_CB = 8192


def _dec(d1, dec_W2, b2_row):
    return pl.pallas_call(
        _dec_body,
        grid=(IN_DIM // _DEC_CB,),
        in_specs=[
            pl.BlockSpec((B, H), lambda j: (0, 0)),
            pl.BlockSpec((H, _DEC_CB), lambda j: (0, j)),
            pl.BlockSpec((1, _DEC_CB), lambda j: (0, j)),
        ],
        out_specs=pl.BlockSpec((B, _DEC_CB), lambda j: (0, j)),
        out_shape=jax.ShapeDtypeStruct((B, IN_DIM), _f32),
    )(d1, dec_W2, b2_row)


def kernel(x, edge_index, W_msg, b_msg, enc_W1, enc_b1, W_mean, b_mean,
           W_logvar, b_logvar, dec_W1, dec_b1, dec_W2, dec_b2):
    x2d = x.reshape(B * N, F)
    src = edge_index[:, 0, :] + (jnp.arange(B, dtype=jnp.int32) * N)[:, None]
    dst = edge_index[:, 1, :]
    src_r = src.reshape(B * NS, NCHUNK, CH)
    dst_r = dst.reshape(B * NS, NCHUNK, CH)

    xbf = x2d.astype(jnp.bfloat16)
    sums2d, cnt = _sc_agg(xbf, src_r, dst_r)
    cnt2d = cnt.reshape(B * N, 1)

    h2d = _conv(sums2d, cnt2d, x2d, W_msg, b_msg.reshape(1, F))
    xs = h2d.reshape(B, IN_DIM)
    h1 = _enc(xs, enc_W1, enc_b1.reshape(1, H))
    eps = jax.random.normal(jax.random.key(42), (B, L), dtype=_f32)
    mean, log_var, d1 = _latent(h1, W_mean, b_mean.reshape(1, L),
                                W_logvar, b_logvar.reshape(1, L),
                                dec_W1, dec_b1.reshape(1, H), eps)
    x_hat = _dec(d1, dec_W2, dec_b2.reshape(1, IN_DIM))
    return (x_hat, mean, log_var)


# speedup vs baseline: 1.1373x; 1.0091x over previous
import functools

import jax
import jax.numpy as jnp
from jax import lax
from jax.experimental import pallas as pl
from jax.experimental.pallas import tpu as pltpu
from jax.experimental.pallas import tpu_sc as plsc

B = 8
N = 4096
F = 64
E = 131072
IN_DIM = N * F
H = 128
L = 64

NC = 2
NS = 16
GPC = B // NC
EPT = E // NS
CH = 128
NCHUNK = EPT // CH
NBUF = 8
LOOK = 4
RPT = N // NS

_f32 = jnp.float32
_bf16 = jnp.bfloat16


def _sc_agg_body(x_hbm, ei_hbm, sums_hbm, cnt_hbm,
                 src_v, dst_v, rb0, rb1, rb2, rb3, rb4, rb5, rb6, rb7,
                 hist_v, zbuf, cacc_v, ctmp_v, acc_sh, cntp_sh,
                 gs0, gs1, gs2, gs3, gs4, gs5, gs6, gs7,
                 ss0, ss1, ss2, ss3, ss4, ss5, ss6, ss7):
    c = lax.axis_index("c")
    s = lax.axis_index("s")
    rbufs = (rb0, rb1, rb2, rb3, rb4, rb5, rb6, rb7)
    gsems = (gs0, gs1, gs2, gs3, gs4, gs5, gs6, gs7)
    ssems = (ss0, ss1, ss2, ss3, ss4, ss5, ss6, ss7)
    zeros16 = jnp.zeros((16,), _f32)
    ones16 = jnp.ones((16,), _f32)
    zeros32 = jnp.zeros((32,), _bf16)

    def _zb(i, _):
        zbuf[i // (F // 32), pl.ds((i % (F // 32)) * 32, 32)] = zeros32
        return 0
    lax.fori_loop(0, RPT * (F // 32), _zb, 0)

    def _round(r, _):
        g = c * GPC + r
        pltpu.sync_copy(zbuf, acc_sh.at[pl.ds(s * RPT, RPT)])

        def _zh(i, __):
            hist_v[pl.ds(i * 16, 16)] = zeros16
            return 0
        lax.fori_loop(0, N // 16, _zh, 0)

        pltpu.sync_copy(ei_hbm.at[g, 0, s], src_v)
        pltpu.sync_copy(ei_hbm.at[g, 1, s], dst_v)
        goff = (g * N).astype(jnp.int32) + jnp.zeros((16,), jnp.int32)

        def _off(i, __):
            sl = pl.ds((i % (CH // 16)) * 16, 16)
            src_v[i // (CH // 16), sl] = src_v[i // (CH // 16), sl] + goff
            return 0
        lax.fori_loop(0, NCHUNK * (CH // 16), _off, 0)
        plsc.subcore_barrier()

        for p in range(LOOK):
            pltpu.async_copy(x_hbm.at[src_v.at[p]], rbufs[p], gsems[p])

        def _wait_scatter(p, j):
            pltpu.make_async_copy(rbufs[p], acc_sh.at[dst_v.at[j]],
                                  ssems[p]).wait()

        def _outer(k, __):
            for p in range(NBUF):
                j = k * NBUF + p
                pltpu.make_async_copy(x_hbm.at[src_v.at[j]], rbufs[p],
                                      gsems[p]).wait()
                for v in range(CH // 16):
                    dvec = dst_v[j, pl.ds(v * 16, 16)]
                    plsc.addupdate_scatter(hist_v, [dvec], ones16)
                pltpu.async_copy(rbufs[p], acc_sh.at[dst_v.at[j]], ssems[p],
                                 add=True)
                j2 = j + LOOK
                p2 = (p + LOOK) % NBUF

                @pl.when(j2 >= NBUF)
                def _():
                    _wait_scatter(p2, j)

                @pl.when(j2 < NCHUNK)
                def _():
                    pltpu.async_copy(x_hbm.at[src_v.at[j2]], rbufs[p2],
                                     gsems[p2])
            return 0
        lax.fori_loop(0, NCHUNK // NBUF, _outer, 0)
        for p in range(LOOK, NBUF):
            _wait_scatter(p, p)

        pltpu.sync_copy(hist_v, cntp_sh.at[s])
        plsc.subcore_barrier()

        pltpu.sync_copy(acc_sh.at[pl.ds(s * RPT, RPT)],
                        sums_hbm.at[pl.ds(g * N + s * RPT, RPT)])
        pltpu.sync_copy(cntp_sh.at[:, pl.ds(s * RPT, RPT)], ctmp_v)
        for q in range(RPT // 16):
            sl = pl.ds(q * 16, 16)
            acc = ctmp_v[0, sl]
            for t in range(1, NS):
                acc = acc + ctmp_v[t, sl]
            cacc_v[sl] = acc
        pltpu.sync_copy(cacc_v, cnt_hbm.at[g * NS + s])
        plsc.subcore_barrier()
        return 0

    lax.fori_loop(0, GPC, _round, 0)


_sc_agg = functools.partial(
    pl.kernel,
    out_type=(jax.ShapeDtypeStruct((B * N, F), _bf16),
              jax.ShapeDtypeStruct((B * NS, RPT), _f32)),
    mesh=plsc.VectorSubcoreMesh(core_axis_name="c", subcore_axis_name="s",
                                num_cores=NC, num_subcores=NS),
    compiler_params=pltpu.CompilerParams(needs_layout_passes=False,
                                         use_tc_tiling_on_sc=False),
    scratch_types=[
        pltpu.VMEM((NCHUNK, CH), jnp.int32),
        pltpu.VMEM((NCHUNK, CH), jnp.int32),
        *([pltpu.VMEM((CH, F), _bf16)] * NBUF),
        pltpu.VMEM((N,), _f32),
        pltpu.VMEM((RPT, F), _bf16),
        pltpu.VMEM((RPT,), _f32),
        pltpu.VMEM((NS, RPT), _f32),
        pltpu.VMEM_SHARED((N, F), _bf16),
        pltpu.VMEM_SHARED((NS, N), _f32),
        *([pltpu.SemaphoreType.DMA] * (2 * NBUF)),
    ],
)(_sc_agg_body)


def _conv_body(sums_ref, cnt_ref, x_ref, wm_ref, bm_ref, out_ref):
    cnt = cnt_ref[...]
    g = sums_ref[...].astype(_f32) / jnp.maximum(cnt, 1.0)
    m = jnp.dot(g, wm_ref[...], preferred_element_type=_f32)
    v = m + bm_ref[...] * (cnt > 0).astype(_f32) + x_ref[...]
    out_ref[...] = jnp.where(v >= 0, v, 0.01 * v)


_CONV_RB = 8192


def _conv(sums2d, cnt2d, x2d, W_msg, b_msg_row):
    rows = B * N
    return pl.pallas_call(
        _conv_body,
        grid=(rows // _CONV_RB,),
        in_specs=[
            pl.BlockSpec((_CONV_RB, F), lambda i: (i, 0)),
            pl.BlockSpec((_CONV_RB, 1), lambda i: (i, 0)),
            pl.BlockSpec((_CONV_RB, F), lambda i: (i, 0)),
            pl.BlockSpec((F, F), lambda i: (0, 0)),
            pl.BlockSpec((1, F), lambda i: (0, 0)),
        ],
        out_specs=pl.BlockSpec((_CONV_RB, F), lambda i: (i, 0)),
        out_shape=jax.ShapeDtypeStruct((rows, F), _f32),
    )(sums2d, cnt2d, x2d, W_msg, b_msg_row)


def _enc_body(xs_ref, w1_ref, b1_ref, out_ref, acc_ref):
    k = pl.program_id(0)

    @pl.when(k == 0)
    def _():
        acc_ref[...] = jnp.zeros_like(acc_ref)

    acc_ref[...] += jnp.dot(xs_ref[...], w1_ref[...],
                            preferred_element_type=_f32)

    @pl.when(k == pl.num_programs(0) - 1)
    def _():
        out_ref[...] = jnp.maximum(acc_ref[...] + b1_ref[...], 0.0)


_ENC_KB = 8192


def _enc(xs, enc_W1, b1_row):
    return pl.pallas_call(
        _enc_body,
        grid=(IN_DIM // _ENC_KB,),
        in_specs=[
            pl.BlockSpec((B, _ENC_KB), lambda k: (0, k)),
            pl.BlockSpec((_ENC_KB, H), lambda k: (k, 0)),
            pl.BlockSpec((1, H), lambda k: (0, 0)),
        ],
        out_specs=pl.BlockSpec((B, H), lambda k: (0, 0)),
        out_shape=jax.ShapeDtypeStruct((B, H), _f32),
        scratch_shapes=[pltpu.VMEM((B, H), _f32)],
    )(xs, enc_W1, b1_row)


def _latent_body(h1_ref, wm_ref, bm_ref, wl_ref, bl_ref, w1_ref, b1_ref,
                 eps_ref, mean_ref, lv_ref, d1_ref):
    h = h1_ref[...]
    mean = jnp.dot(h, wm_ref[...], preferred_element_type=_f32) + bm_ref[...]
    lv = jnp.dot(h, wl_ref[...], preferred_element_type=_f32) + bl_ref[...]
    z = mean + jnp.exp(0.5 * lv) * eps_ref[...]
    d1 = jnp.dot(z, w1_ref[...], preferred_element_type=_f32) + b1_ref[...]
    mean_ref[...] = mean
    lv_ref[...] = lv
    d1_ref[...] = jnp.maximum(d1, 0.0)


def _latent(h1, W_mean, bm_row, W_logvar, bl_row, dec_W1, db1_row, eps):
    return pl.pallas_call(
        _latent_body,
        out_shape=(jax.ShapeDtypeStruct((B, L), _f32),
                   jax.ShapeDtypeStruct((B, L), _f32),
                   jax.ShapeDtypeStruct((B, H), _f32)),
    )(h1, W_mean, bm_row, W_logvar, bl_row, dec_W1, db1_row, eps)


def _dec_body(d1_ref, w2_ref, b2_ref, out_ref):
    y = jnp.dot(d1_ref[...], w2_ref[...], preferred_element_type=_f32)
    out_ref[...] = jax.nn.sigmoid(y + b2_ref[...])


_DEC_CB = 8192


def _dec(d1, dec_W2, b2_row):
    return pl.pallas_call(
        _dec_body,
        grid=(IN_DIM // _DEC_CB,),
        in_specs=[
            pl.BlockSpec((B, H), lambda j: (0, 0)),
            pl.BlockSpec((H, _DEC_CB), lambda j: (0, j)),
            pl.BlockSpec((1, _DEC_CB), lambda j: (0, j)),
        ],
        out_specs=pl.BlockSpec((B, _DEC_CB), lambda j: (0, j)),
        out_shape=jax.ShapeDtypeStruct((B, IN_DIM), _f32),
    )(d1, dec_W2, b2_row)


def kernel(x, edge_index, W_msg, b_msg, enc_W1, enc_b1, W_mean, b_mean,
           W_logvar, b_logvar, dec_W1, dec_b1, dec_W2, dec_b2):
    x2d = x.reshape(B * N, F)
    ei_r = edge_index.reshape(B, 2, NS, NCHUNK, CH)

    xbf = x2d.astype(jnp.bfloat16)
    sums2d, cnt = _sc_agg(xbf, ei_r)
    cnt2d = cnt.reshape(B * N, 1)

    h2d = _conv(sums2d, cnt2d, x2d, W_msg, b_msg.reshape(1, F))
    xs = h2d.reshape(B, IN_DIM)
    h1 = _enc(xs, enc_W1, enc_b1.reshape(1, H))
    eps = jax.random.normal(jax.random.key(42), (B, L), dtype=_f32)
    mean, log_var, d1 = _latent(h1, W_mean, b_mean.reshape(1, L),
                                W_logvar, b_logvar.reshape(1, L),
                                dec_W1, dec_b1.reshape(1, H), eps)
    x_hat = _dec(d1, dec_W2, dec_b2.reshape(1, IN_DIM))
    return (x_hat, mean, log_var)


# speedup vs baseline: 1.1457x; 1.0074x over previous
import functools

import jax
import jax.numpy as jnp
from jax import lax
from jax.experimental import pallas as pl
from jax.experimental.pallas import tpu as pltpu
from jax.experimental.pallas import tpu_sc as plsc

B = 8
N = 4096
F = 64
E = 131072
IN_DIM = N * F
H = 128
L = 64

NC = 2
NS = 16
GPC = B // NC
EPT = E // NS
CH = 128
NCHUNK = EPT // CH
NBUF = 8
LOOK = 4
RPT = N // NS

_f32 = jnp.float32
_bf16 = jnp.bfloat16


def _sc_agg_body(x_hbm, ei_hbm, sums_hbm, cnt_hbm,
                 src_v, dst_v, rb0, rb1, rb2, rb3, rb4, rb5, rb6, rb7,
                 hist_v, zbuf, cacc_v, ctmp_v, acc_sh, cntp_sh,
                 gs0, gs1, gs2, gs3, gs4, gs5, gs6, gs7,
                 ss0, ss1, ss2, ss3, ss4, ss5, ss6, ss7):
    c = lax.axis_index("c")
    s = lax.axis_index("s")
    rbufs = (rb0, rb1, rb2, rb3, rb4, rb5, rb6, rb7)
    gsems = (gs0, gs1, gs2, gs3, gs4, gs5, gs6, gs7)
    ssems = (ss0, ss1, ss2, ss3, ss4, ss5, ss6, ss7)
    zeros16 = jnp.zeros((16,), _f32)
    ones16 = jnp.ones((16,), _f32)
    zeros32 = jnp.zeros((32,), _bf16)

    def _zb(i, _):
        zbuf[i // (F // 32), pl.ds((i % (F // 32)) * 32, 32)] = zeros32
        return 0
    lax.fori_loop(0, RPT * (F // 32), _zb, 0)

    def _round(r, _):
        g = c * GPC + r
        pltpu.sync_copy(zbuf, acc_sh.at[pl.ds(s * RPT, RPT)])

        def _zh(i, __):
            hist_v[pl.ds(i * 16, 16)] = zeros16
            return 0
        lax.fori_loop(0, N // 16, _zh, 0)

        pltpu.sync_copy(ei_hbm.at[g, 0, s], src_v)
        pltpu.sync_copy(ei_hbm.at[g, 1, s], dst_v)
        goff = (g * N).astype(jnp.int32) + jnp.zeros((16,), jnp.int32)

        def _off(i, __):
            sl = pl.ds((i % (CH // 16)) * 16, 16)
            src_v[i // (CH // 16), sl] = src_v[i // (CH // 16), sl] + goff
            return 0
        lax.fori_loop(0, NCHUNK * (CH // 16), _off, 0)
        plsc.subcore_barrier()

        for p in range(LOOK):
            pltpu.async_copy(x_hbm.at[src_v.at[p]], rbufs[p], gsems[p])

        def _wait_scatter(p, j):
            pltpu.make_async_copy(rbufs[p], acc_sh.at[dst_v.at[j]],
                                  ssems[p]).wait()

        def _outer(k, __):
            for p in range(NBUF):
                j = k * NBUF + p
                pltpu.make_async_copy(x_hbm.at[src_v.at[j]], rbufs[p],
                                      gsems[p]).wait()
                for v in range(CH // 16):
                    dvec = dst_v[j, pl.ds(v * 16, 16)]
                    plsc.addupdate_scatter(hist_v, [dvec], ones16)
                pltpu.async_copy(rbufs[p], acc_sh.at[dst_v.at[j]], ssems[p],
                                 add=True)
                j2 = j + LOOK
                p2 = (p + LOOK) % NBUF

                @pl.when(j2 >= NBUF)
                def _():
                    _wait_scatter(p2, j)

                @pl.when(j2 < NCHUNK)
                def _():
                    pltpu.async_copy(x_hbm.at[src_v.at[j2]], rbufs[p2],
                                     gsems[p2])
            return 0
        lax.fori_loop(0, NCHUNK // NBUF, _outer, 0)
        for p in range(LOOK, NBUF):
            _wait_scatter(p, p)

        pltpu.sync_copy(hist_v, cntp_sh.at[s])
        plsc.subcore_barrier()

        pltpu.sync_copy(acc_sh.at[pl.ds(s * RPT, RPT)],
                        sums_hbm.at[pl.ds(g * N + s * RPT, RPT)])
        pltpu.sync_copy(cntp_sh.at[:, pl.ds(s * RPT, RPT)], ctmp_v)
        for q in range(RPT // 16):
            sl = pl.ds(q * 16, 16)
            acc = ctmp_v[0, sl]
            for t in range(1, NS):
                acc = acc + ctmp_v[t, sl]
            cacc_v[sl] = acc
        pltpu.sync_copy(cacc_v, cnt_hbm.at[g * NS + s])
        plsc.subcore_barrier()
        return 0

    lax.fori_loop(0, GPC, _round, 0)


_sc_agg = functools.partial(
    pl.kernel,
    out_type=(jax.ShapeDtypeStruct((B * N, F), _bf16),
              jax.ShapeDtypeStruct((B * NS, RPT), _f32)),
    mesh=plsc.VectorSubcoreMesh(core_axis_name="c", subcore_axis_name="s",
                                num_cores=NC, num_subcores=NS),
    compiler_params=pltpu.CompilerParams(needs_layout_passes=False,
                                         use_tc_tiling_on_sc=False),
    scratch_types=[
        pltpu.VMEM((NCHUNK, CH), jnp.int32),
        pltpu.VMEM((NCHUNK, CH), jnp.int32),
        *([pltpu.VMEM((CH, F), _bf16)] * NBUF),
        pltpu.VMEM((N,), _f32),
        pltpu.VMEM((RPT, F), _bf16),
        pltpu.VMEM((RPT,), _f32),
        pltpu.VMEM((NS, RPT), _f32),
        pltpu.VMEM_SHARED((N, F), _bf16),
        pltpu.VMEM_SHARED((NS, N), _f32),
        *([pltpu.SemaphoreType.DMA] * (2 * NBUF)),
    ],
)(_sc_agg_body)


def _conv_body(sums_ref, cnt_ref, x_ref, wm_ref, bm_ref, out_ref):
    cnt = cnt_ref[...]
    g = sums_ref[...].astype(_f32) / jnp.maximum(cnt, 1.0)
    m = jnp.dot(g, wm_ref[...], preferred_element_type=_f32)
    v = m + bm_ref[...] * (cnt > 0).astype(_f32) + x_ref[...]
    out_ref[...] = jnp.where(v >= 0, v, 0.01 * v).astype(_bf16)


_CONV_RB = 8192


def _conv(sums2d, cnt2d, x2d, W_msg, b_msg_row):
    rows = B * N
    return pl.pallas_call(
        _conv_body,
        grid=(rows // _CONV_RB,),
        in_specs=[
            pl.BlockSpec((_CONV_RB, F), lambda i: (i, 0)),
            pl.BlockSpec((_CONV_RB, 1), lambda i: (i, 0)),
            pl.BlockSpec((_CONV_RB, F), lambda i: (i, 0)),
            pl.BlockSpec((F, F), lambda i: (0, 0)),
            pl.BlockSpec((1, F), lambda i: (0, 0)),
        ],
        out_specs=pl.BlockSpec((_CONV_RB, F), lambda i: (i, 0)),
        out_shape=jax.ShapeDtypeStruct((rows, F), _bf16),
    )(sums2d, cnt2d, x2d, W_msg, b_msg_row)


def _enc_body(xs_ref, w1_ref, b1_ref, wm_ref, bm_ref, wl_ref, bl_ref,
              wd_ref, bd_ref, eps_ref, mean_ref, lv_ref, d1_ref, acc_ref):
    k = pl.program_id(0)

    @pl.when(k == 0)
    def _():
        acc_ref[...] = jnp.zeros_like(acc_ref)

    acc_ref[...] += jnp.dot(xs_ref[...].astype(_f32), w1_ref[...],
                            preferred_element_type=_f32)

    @pl.when(k == pl.num_programs(0) - 1)
    def _():
        h = jnp.maximum(acc_ref[...] + b1_ref[...], 0.0)
        mean = jnp.dot(h, wm_ref[...], preferred_element_type=_f32) \
            + bm_ref[...]
        lv = jnp.dot(h, wl_ref[...], preferred_element_type=_f32) \
            + bl_ref[...]
        z = mean + jnp.exp(0.5 * lv) * eps_ref[...]
        d1 = jnp.dot(z, wd_ref[...], preferred_element_type=_f32) \
            + bd_ref[...]
        mean_ref[...] = mean
        lv_ref[...] = lv
        d1_ref[...] = jnp.maximum(d1, 0.0)


_ENC_KB = 8192


def _enc(xs, enc_W1, b1_row, W_mean, bm_row, W_logvar, bl_row,
         dec_W1, db1_row, eps):
    full = lambda a, b: pl.BlockSpec((a, b), lambda k: (0, 0))
    return pl.pallas_call(
        _enc_body,
        grid=(IN_DIM // _ENC_KB,),
        in_specs=[
            pl.BlockSpec((B, _ENC_KB), lambda k: (0, k)),
            pl.BlockSpec((_ENC_KB, H), lambda k: (k, 0)),
            full(1, H), full(H, L), full(1, L), full(H, L), full(1, L),
            full(L, H), full(1, H), full(B, L),
        ],
        out_specs=(full(B, L), full(B, L), full(B, H)),
        out_shape=(jax.ShapeDtypeStruct((B, L), _f32),
                   jax.ShapeDtypeStruct((B, L), _f32),
                   jax.ShapeDtypeStruct((B, H), _f32)),
        scratch_shapes=[pltpu.VMEM((B, H), _f32)],
    )(xs, enc_W1, b1_row, W_mean, bm_row, W_logvar, bl_row,
      dec_W1, db1_row, eps)


def _dec_body(d1_ref, w2_ref, b2_ref, out_ref):
    y = jnp.dot(d1_ref[...], w2_ref[...], preferred_element_type=_f32)
    out_ref[...] = jax.nn.sigmoid(y + b2_ref[...])


_DEC_CB = 8192


def _dec(d1, dec_W2, b2_row):
    return pl.pallas_call(
        _dec_body,
        grid=(IN_DIM // _DEC_CB,),
        in_specs=[
            pl.BlockSpec((B, H), lambda j: (0, 0)),
            pl.BlockSpec((H, _DEC_CB), lambda j: (0, j)),
            pl.BlockSpec((1, _DEC_CB), lambda j: (0, j)),
        ],
        out_specs=pl.BlockSpec((B, _DEC_CB), lambda j: (0, j)),
        out_shape=jax.ShapeDtypeStruct((B, IN_DIM), _f32),
    )(d1, dec_W2, b2_row)


def kernel(x, edge_index, W_msg, b_msg, enc_W1, enc_b1, W_mean, b_mean,
           W_logvar, b_logvar, dec_W1, dec_b1, dec_W2, dec_b2):
    x2d = x.reshape(B * N, F)
    ei_r = edge_index.reshape(B, 2, NS, NCHUNK, CH)

    xbf = x2d.astype(jnp.bfloat16)
    sums2d, cnt = _sc_agg(xbf, ei_r)
    cnt2d = cnt.reshape(B * N, 1)

    h2d = _conv(sums2d, cnt2d, x2d, W_msg, b_msg.reshape(1, F))
    xs = h2d.reshape(B, IN_DIM)
    eps = jax.random.normal(jax.random.key(42), (B, L), dtype=_f32)
    mean, log_var, d1 = _enc(xs, enc_W1, enc_b1.reshape(1, H),
                             W_mean, b_mean.reshape(1, L),
                             W_logvar, b_logvar.reshape(1, L),
                             dec_W1, dec_b1.reshape(1, H), eps)
    x_hat = _dec(d1, dec_W2, dec_b2.reshape(1, IN_DIM))
    return (x_hat, mean, log_var)


# speedup vs baseline: 1.1667x; 1.0184x over previous
import functools

import jax
import jax.numpy as jnp
from jax import lax
from jax.experimental import pallas as pl
from jax.experimental.pallas import tpu as pltpu
from jax.experimental.pallas import tpu_sc as plsc

B = 8
N = 4096
F = 64
E = 131072
IN_DIM = N * F
H = 128
L = 64

NC = 2
NS = 16
GPC = B // NC
EPT = E // NS
CH = 128
NCHUNK = EPT // CH
NBUF = 8
LOOK = 4
RPT = N // NS

_f32 = jnp.float32
_bf16 = jnp.bfloat16


def _sc_agg_body(x_hbm, ei_hbm, sums_hbm, cnt_hbm,
                 src_v, dst_v, rb0, rb1, rb2, rb3, rb4, rb5, rb6, rb7,
                 hist_v, zbuf, cacc_v, ctmp_v, acc_sh, cntp_sh,
                 gs0, gs1, gs2, gs3, gs4, gs5, gs6, gs7,
                 ss0, ss1, ss2, ss3, ss4, ss5, ss6, ss7):
    c = lax.axis_index("c")
    s = lax.axis_index("s")
    rbufs = (rb0, rb1, rb2, rb3, rb4, rb5, rb6, rb7)
    gsems = (gs0, gs1, gs2, gs3, gs4, gs5, gs6, gs7)
    ssems = (ss0, ss1, ss2, ss3, ss4, ss5, ss6, ss7)
    zeros16 = jnp.zeros((16,), _f32)
    ones16 = jnp.ones((16,), _f32)
    zeros32 = jnp.zeros((32,), _bf16)

    def _zb(i, _):
        zbuf[i // (F // 32), pl.ds((i % (F // 32)) * 32, 32)] = zeros32
        return 0
    lax.fori_loop(0, RPT * (F // 32), _zb, 0)

    for r0 in range(GPC):
        pltpu.sync_copy(zbuf, acc_sh.at[pl.ds(r0 * N + s * RPT, RPT)])

    def _zh(i, __):
        hist_v[pl.ds(i * 16, 16)] = zeros16
        return 0
    lax.fori_loop(0, GPC * N // 16, _zh, 0)
    plsc.subcore_barrier()

    def _round(r, _):
        g = c * GPC + r

        pltpu.sync_copy(ei_hbm.at[g, 0, s], src_v)
        pltpu.sync_copy(ei_hbm.at[g, 1, s], dst_v)
        goff = (g * N).astype(jnp.int32) + jnp.zeros((16,), jnp.int32)
        roff = (r * N).astype(jnp.int32) + jnp.zeros((16,), jnp.int32)

        def _off(i, __):
            j = i // (CH // 16)
            sl = pl.ds((i % (CH // 16)) * 16, 16)
            src_v[j, sl] = src_v[j, sl] + goff
            dst_v[j, sl] = dst_v[j, sl] + roff
            return 0
        lax.fori_loop(0, NCHUNK * (CH // 16), _off, 0)

        for p in range(LOOK):
            pltpu.async_copy(x_hbm.at[src_v.at[p]], rbufs[p], gsems[p])

        def _wait_scatter(p, j):
            pltpu.make_async_copy(rbufs[p], acc_sh.at[dst_v.at[j]],
                                  ssems[p]).wait()

        def _outer(k, __):
            for p in range(NBUF):
                j = k * NBUF + p
                pltpu.make_async_copy(x_hbm.at[src_v.at[j]], rbufs[p],
                                      gsems[p]).wait()
                for v in range(CH // 16):
                    dvec = dst_v[j, pl.ds(v * 16, 16)]
                    plsc.addupdate_scatter(hist_v, [dvec], ones16)
                pltpu.async_copy(rbufs[p], acc_sh.at[dst_v.at[j]], ssems[p],
                                 add=True)
                j2 = j + LOOK
                p2 = (p + LOOK) % NBUF

                @pl.when(j2 >= NBUF)
                def _():
                    _wait_scatter(p2, j)

                @pl.when(j2 < NCHUNK)
                def _():
                    pltpu.async_copy(x_hbm.at[src_v.at[j2]], rbufs[p2],
                                     gsems[p2])
            return 0
        lax.fori_loop(0, NCHUNK // NBUF, _outer, 0)
        for p in range(LOOK, NBUF):
            _wait_scatter(p, p)

        return 0

    lax.fori_loop(0, GPC, _round, 0)

    def _pub(r, _):
        pltpu.sync_copy(hist_v.at[pl.ds(r * N, N)], cntp_sh.at[r, s])
        return 0
    lax.fori_loop(0, GPC, _pub, 0)
    plsc.subcore_barrier()

    def _export(r, _):
        g = c * GPC + r
        pltpu.sync_copy(acc_sh.at[pl.ds(r * N + s * RPT, RPT)],
                        sums_hbm.at[pl.ds(g * N + s * RPT, RPT)])
        pltpu.sync_copy(cntp_sh.at[r, :, pl.ds(s * RPT, RPT)], ctmp_v)
        for q in range(RPT // 16):
            sl = pl.ds(q * 16, 16)
            acc = ctmp_v[0, sl]
            for t in range(1, NS):
                acc = acc + ctmp_v[t, sl]
            cacc_v[sl] = acc
        pltpu.sync_copy(cacc_v, cnt_hbm.at[g * NS + s])
        return 0
    lax.fori_loop(0, GPC, _export, 0)


_sc_agg = functools.partial(
    pl.kernel,
    out_type=(jax.ShapeDtypeStruct((B * N, F), _bf16),
              jax.ShapeDtypeStruct((B * NS, RPT), _f32)),
    mesh=plsc.VectorSubcoreMesh(core_axis_name="c", subcore_axis_name="s",
                                num_cores=NC, num_subcores=NS),
    compiler_params=pltpu.CompilerParams(needs_layout_passes=False,
                                         use_tc_tiling_on_sc=False),
    scratch_types=[
        pltpu.VMEM((NCHUNK, CH), jnp.int32),
        pltpu.VMEM((NCHUNK, CH), jnp.int32),
        *([pltpu.VMEM((CH, F), _bf16)] * NBUF),
        pltpu.VMEM((GPC * N,), _f32),
        pltpu.VMEM((RPT, F), _bf16),
        pltpu.VMEM((RPT,), _f32),
        pltpu.VMEM((NS, RPT), _f32),
        pltpu.VMEM_SHARED((GPC * N, F), _bf16),
        pltpu.VMEM_SHARED((GPC, NS, N), _f32),
        *([pltpu.SemaphoreType.DMA] * (2 * NBUF)),
    ],
)(_sc_agg_body)


def _conv_body(sums_ref, cnt_ref, x_ref, wm_ref, bm_ref, out_ref):
    cnt = cnt_ref[...]
    g = sums_ref[...].astype(_f32) / jnp.maximum(cnt, 1.0)
    m = jnp.dot(g, wm_ref[...], preferred_element_type=_f32)
    v = m + bm_ref[...] * (cnt > 0).astype(_f32) + x_ref[...]
    out_ref[...] = jnp.where(v >= 0, v, 0.01 * v).astype(_bf16)


_CONV_RB = 8192


def _conv(sums2d, cnt2d, x2d, W_msg, b_msg_row):
    rows = B * N
    return pl.pallas_call(
        _conv_body,
        grid=(rows // _CONV_RB,),
        in_specs=[
            pl.BlockSpec((_CONV_RB, F), lambda i: (i, 0)),
            pl.BlockSpec((_CONV_RB, 1), lambda i: (i, 0)),
            pl.BlockSpec((_CONV_RB, F), lambda i: (i, 0)),
            pl.BlockSpec((F, F), lambda i: (0, 0)),
            pl.BlockSpec((1, F), lambda i: (0, 0)),
        ],
        out_specs=pl.BlockSpec((_CONV_RB, F), lambda i: (i, 0)),
        out_shape=jax.ShapeDtypeStruct((rows, F), _bf16),
    )(sums2d, cnt2d, x2d, W_msg, b_msg_row)


def _enc_body(xs_ref, w1_ref, b1_ref, wm_ref, bm_ref, wl_ref, bl_ref,
              wd_ref, bd_ref, eps_ref, mean_ref, lv_ref, d1_ref, acc_ref):
    k = pl.program_id(0)

    @pl.when(k == 0)
    def _():
        acc_ref[...] = jnp.zeros_like(acc_ref)

    acc_ref[...] += jnp.dot(xs_ref[...].astype(_f32), w1_ref[...],
                            preferred_element_type=_f32)

    @pl.when(k == pl.num_programs(0) - 1)
    def _():
        h = jnp.maximum(acc_ref[...] + b1_ref[...], 0.0)
        mean = jnp.dot(h, wm_ref[...], preferred_element_type=_f32) \
            + bm_ref[...]
        lv = jnp.dot(h, wl_ref[...], preferred_element_type=_f32) \
            + bl_ref[...]
        z = mean + jnp.exp(0.5 * lv) * eps_ref[...]
        d1 = jnp.dot(z, wd_ref[...], preferred_element_type=_f32) \
            + bd_ref[...]
        mean_ref[...] = mean
        lv_ref[...] = lv
        d1_ref[...] = jnp.maximum(d1, 0.0)


_ENC_KB = 8192


def _enc(xs, enc_W1, b1_row, W_mean, bm_row, W_logvar, bl_row,
         dec_W1, db1_row, eps):
    full = lambda a, b: pl.BlockSpec((a, b), lambda k: (0, 0))
    return pl.pallas_call(
        _enc_body,
        grid=(IN_DIM // _ENC_KB,),
        in_specs=[
            pl.BlockSpec((B, _ENC_KB), lambda k: (0, k)),
            pl.BlockSpec((_ENC_KB, H), lambda k: (k, 0)),
            full(1, H), full(H, L), full(1, L), full(H, L), full(1, L),
            full(L, H), full(1, H), full(B, L),
        ],
        out_specs=(full(B, L), full(B, L), full(B, H)),
        out_shape=(jax.ShapeDtypeStruct((B, L), _f32),
                   jax.ShapeDtypeStruct((B, L), _f32),
                   jax.ShapeDtypeStruct((B, H), _f32)),
        scratch_shapes=[pltpu.VMEM((B, H), _f32)],
    )(xs, enc_W1, b1_row, W_mean, bm_row, W_logvar, bl_row,
      dec_W1, db1_row, eps)


def _dec_body(d1_ref, w2_ref, b2_ref, out_ref):
    y = jnp.dot(d1_ref[...], w2_ref[...], preferred_element_type=_f32)
    out_ref[...] = jax.nn.sigmoid(y + b2_ref[...])


_DEC_CB = 8192


def _dec(d1, dec_W2, b2_row):
    return pl.pallas_call(
        _dec_body,
        grid=(IN_DIM // _DEC_CB,),
        in_specs=[
            pl.BlockSpec((B, H), lambda j: (0, 0)),
            pl.BlockSpec((H, _DEC_CB), lambda j: (0, j)),
            pl.BlockSpec((1, _DEC_CB), lambda j: (0, j)),
        ],
        out_specs=pl.BlockSpec((B, _DEC_CB), lambda j: (0, j)),
        out_shape=jax.ShapeDtypeStruct((B, IN_DIM), _f32),
    )(d1, dec_W2, b2_row)


def kernel(x, edge_index, W_msg, b_msg, enc_W1, enc_b1, W_mean, b_mean,
           W_logvar, b_logvar, dec_W1, dec_b1, dec_W2, dec_b2):
    x2d = x.reshape(B * N, F)
    ei_r = edge_index.reshape(B, 2, NS, NCHUNK, CH)

    xbf = x2d.astype(jnp.bfloat16)
    sums2d, cnt = _sc_agg(xbf, ei_r)
    cnt2d = cnt.reshape(B * N, 1)

    h2d = _conv(sums2d, cnt2d, x2d, W_msg, b_msg.reshape(1, F))
    xs = h2d.reshape(B, IN_DIM)
    eps = jax.random.normal(jax.random.key(42), (B, L), dtype=_f32)
    mean, log_var, d1 = _enc(xs, enc_W1, enc_b1.reshape(1, H),
                             W_mean, b_mean.reshape(1, L),
                             W_logvar, b_logvar.reshape(1, L),
                             dec_W1, dec_b1.reshape(1, H), eps)
    x_hat = _dec(d1, dec_W2, dec_b2.reshape(1, IN_DIM))
    return (x_hat, mean, log_var)


# speedup vs baseline: 1.2061x; 1.0338x over previous
import functools

import jax
import jax.numpy as jnp
from jax import lax
from jax.experimental import pallas as pl
from jax.experimental.pallas import tpu as pltpu
from jax.experimental.pallas import tpu_sc as plsc

B = 8
N = 4096
F = 64
E = 131072
IN_DIM = N * F
H = 128
L = 64

NC = 2
NS = 16
GPC = B // NC
EPT = E // NS
CH = 128
NCHUNK = EPT // CH
NBUF = 8
LOOK = 4
RPT = N // NS

_f32 = jnp.float32
_bf16 = jnp.bfloat16


def _sc_agg_body(x_hbm, ei_hbm, sums_hbm, cnt_hbm,
                 src_v, dst_v, rb0, rb1, rb2, rb3, rb4, rb5, rb6, rb7,
                 hist_v, zbuf, cacc_v, ctmp_v, acc_sh, cntp_sh,
                 gs0, gs1, gs2, gs3, gs4, gs5, gs6, gs7,
                 ss0, ss1, ss2, ss3, ss4, ss5, ss6, ss7):
    c = lax.axis_index("c")
    s = lax.axis_index("s")
    rbufs = (rb0, rb1, rb2, rb3, rb4, rb5, rb6, rb7)
    gsems = (gs0, gs1, gs2, gs3, gs4, gs5, gs6, gs7)
    ssems = (ss0, ss1, ss2, ss3, ss4, ss5, ss6, ss7)
    zeros16 = jnp.zeros((16,), _f32)
    ones16 = jnp.ones((16,), _f32)
    zeros32 = jnp.zeros((32,), _bf16)

    def _zb(i, _):
        zbuf[i // (F // 32), pl.ds((i % (F // 32)) * 32, 32)] = zeros32
        return 0
    lax.fori_loop(0, RPT * (F // 32), _zb, 0)

    for r0 in range(GPC):
        pltpu.sync_copy(zbuf, acc_sh.at[pl.ds(r0 * N + s * RPT, RPT)])

    def _zh(i, __):
        hist_v[pl.ds(i * 16, 16)] = zeros16
        return 0
    lax.fori_loop(0, GPC * N // 16, _zh, 0)
    plsc.subcore_barrier()

    def _round(r, _):
        g = c * GPC + r

        cp_s = pltpu.async_copy(ei_hbm.at[g, 0, s], src_v, gs0)
        cp_d = pltpu.async_copy(ei_hbm.at[g, 1, s], dst_v, gs1)
        cp_s.wait()
        cp_d.wait()
        roff = (r * N).astype(jnp.int32) + jnp.zeros((16,), jnp.int32)

        def _off(i, __):
            j = i // (CH // 16)
            sl = pl.ds((i % (CH // 16)) * 16, 16)
            dst_v[j, sl] = dst_v[j, sl] + roff
            return 0
        lax.fori_loop(0, NCHUNK * (CH // 16), _off, 0)

        for p in range(LOOK):
            pltpu.async_copy(x_hbm.at[g].at[src_v.at[p]], rbufs[p], gsems[p])

        def _wait_scatter(p, j):
            pltpu.make_async_copy(rbufs[p], acc_sh.at[dst_v.at[j]],
                                  ssems[p]).wait()

        def _outer(k, __):
            for p in range(NBUF):
                j = k * NBUF + p
                pltpu.make_async_copy(x_hbm.at[g].at[src_v.at[j]], rbufs[p],
                                      gsems[p]).wait()
                for v in range(CH // 16):
                    dvec = dst_v[j, pl.ds(v * 16, 16)]
                    plsc.addupdate_scatter(hist_v, [dvec], ones16)
                pltpu.async_copy(rbufs[p], acc_sh.at[dst_v.at[j]], ssems[p],
                                 add=True)
                j2 = j + LOOK
                p2 = (p + LOOK) % NBUF

                @pl.when(j2 >= NBUF)
                def _():
                    _wait_scatter(p2, j)

                @pl.when(j2 < NCHUNK)
                def _():
                    pltpu.async_copy(x_hbm.at[g].at[src_v.at[j2]], rbufs[p2],
                                     gsems[p2])
            return 0
        lax.fori_loop(0, NCHUNK // NBUF, _outer, 0)
        for p in range(LOOK, NBUF):
            _wait_scatter(p, p)

        return 0

    lax.fori_loop(0, GPC, _round, 0)

    def _pub(r, _):
        pltpu.sync_copy(hist_v.at[pl.ds(r * N, N)], cntp_sh.at[r, s])
        return 0
    lax.fori_loop(0, GPC, _pub, 0)
    plsc.subcore_barrier()

    def _export(r, _):
        g = c * GPC + r
        pltpu.sync_copy(acc_sh.at[pl.ds(r * N + s * RPT, RPT)],
                        sums_hbm.at[pl.ds(g * N + s * RPT, RPT)])
        pltpu.sync_copy(cntp_sh.at[r, :, pl.ds(s * RPT, RPT)], ctmp_v)
        for q in range(RPT // 16):
            sl = pl.ds(q * 16, 16)
            acc = ctmp_v[0, sl]
            for t in range(1, NS):
                acc = acc + ctmp_v[t, sl]
            cacc_v[sl] = acc
        pltpu.sync_copy(cacc_v, cnt_hbm.at[g * NS + s])
        return 0
    lax.fori_loop(0, GPC, _export, 0)


_sc_agg = functools.partial(
    pl.kernel,
    out_type=(jax.ShapeDtypeStruct((B * N, F), _bf16),
              jax.ShapeDtypeStruct((B * NS, RPT), _f32)),
    mesh=plsc.VectorSubcoreMesh(core_axis_name="c", subcore_axis_name="s",
                                num_cores=NC, num_subcores=NS),
    compiler_params=pltpu.CompilerParams(needs_layout_passes=False,
                                         use_tc_tiling_on_sc=False),
    scratch_types=[
        pltpu.VMEM((NCHUNK, CH), jnp.int32),
        pltpu.VMEM((NCHUNK, CH), jnp.int32),
        *([pltpu.VMEM((CH, F), _bf16)] * NBUF),
        pltpu.VMEM((GPC * N,), _f32),
        pltpu.VMEM((RPT, F), _bf16),
        pltpu.VMEM((RPT,), _f32),
        pltpu.VMEM((NS, RPT), _f32),
        pltpu.VMEM_SHARED((GPC * N, F), _bf16),
        pltpu.VMEM_SHARED((GPC, NS, N), _f32),
        *([pltpu.SemaphoreType.DMA] * (2 * NBUF)),
    ],
)(_sc_agg_body)


def _conv_body(sums_ref, cnt_ref, x_ref, wm_ref, bm_ref, out_ref):
    cnt = cnt_ref[...]
    g = sums_ref[...].astype(_f32) / jnp.maximum(cnt, 1.0)
    m = jnp.dot(g, wm_ref[...], preferred_element_type=_f32)
    v = m + bm_ref[...] * (cnt > 0).astype(_f32) + x_ref[...]
    out_ref[...] = jnp.where(v >= 0, v, 0.01 * v).astype(_bf16)


_CONV_RB = 8192


def _conv(sums2d, cnt2d, x2d, W_msg, b_msg_row):
    rows = B * N
    return pl.pallas_call(
        _conv_body,
        grid=(rows // _CONV_RB,),
        in_specs=[
            pl.BlockSpec((_CONV_RB, F), lambda i: (i, 0)),
            pl.BlockSpec((_CONV_RB, 1), lambda i: (i, 0)),
            pl.BlockSpec((_CONV_RB, F), lambda i: (i, 0)),
            pl.BlockSpec((F, F), lambda i: (0, 0)),
            pl.BlockSpec((1, F), lambda i: (0, 0)),
        ],
        out_specs=pl.BlockSpec((_CONV_RB, F), lambda i: (i, 0)),
        out_shape=jax.ShapeDtypeStruct((rows, F), _bf16),
    )(sums2d, cnt2d, x2d, W_msg, b_msg_row)


def _enc_body(xs_ref, w1_ref, b1_ref, wm_ref, bm_ref, wl_ref, bl_ref,
              wd_ref, bd_ref, eps_ref, mean_ref, lv_ref, d1_ref, acc_ref):
    k = pl.program_id(0)

    @pl.when(k == 0)
    def _():
        acc_ref[...] = jnp.zeros_like(acc_ref)

    acc_ref[...] += jnp.dot(xs_ref[...].astype(_f32), w1_ref[...],
                            preferred_element_type=_f32)

    @pl.when(k == pl.num_programs(0) - 1)
    def _():
        h = jnp.maximum(acc_ref[...] + b1_ref[...], 0.0)
        mean = jnp.dot(h, wm_ref[...], preferred_element_type=_f32) \
            + bm_ref[...]
        lv = jnp.dot(h, wl_ref[...], preferred_element_type=_f32) \
            + bl_ref[...]
        z = mean + jnp.exp(0.5 * lv) * eps_ref[...]
        d1 = jnp.dot(z, wd_ref[...], preferred_element_type=_f32) \
            + bd_ref[...]
        mean_ref[...] = mean
        lv_ref[...] = lv
        d1_ref[...] = jnp.maximum(d1, 0.0)


_ENC_KB = 8192


def _enc(xs, enc_W1, b1_row, W_mean, bm_row, W_logvar, bl_row,
         dec_W1, db1_row, eps):
    full = lambda a, b: pl.BlockSpec((a, b), lambda k: (0, 0))
    return pl.pallas_call(
        _enc_body,
        grid=(IN_DIM // _ENC_KB,),
        in_specs=[
            pl.BlockSpec((B, _ENC_KB), lambda k: (0, k)),
            pl.BlockSpec((_ENC_KB, H), lambda k: (k, 0)),
            full(1, H), full(H, L), full(1, L), full(H, L), full(1, L),
            full(L, H), full(1, H), full(B, L),
        ],
        out_specs=(full(B, L), full(B, L), full(B, H)),
        out_shape=(jax.ShapeDtypeStruct((B, L), _f32),
                   jax.ShapeDtypeStruct((B, L), _f32),
                   jax.ShapeDtypeStruct((B, H), _f32)),
        scratch_shapes=[pltpu.VMEM((B, H), _f32)],
    )(xs, enc_W1, b1_row, W_mean, bm_row, W_logvar, bl_row,
      dec_W1, db1_row, eps)


def _dec_body(d1_ref, w2_ref, b2_ref, out_ref):
    y = jnp.dot(d1_ref[...], w2_ref[...], preferred_element_type=_f32)
    out_ref[...] = jax.nn.sigmoid(y + b2_ref[...])


_DEC_CB = 8192


def _dec(d1, dec_W2, b2_row):
    return pl.pallas_call(
        _dec_body,
        grid=(IN_DIM // _DEC_CB,),
        in_specs=[
            pl.BlockSpec((B, H), lambda j: (0, 0)),
            pl.BlockSpec((H, _DEC_CB), lambda j: (0, j)),
            pl.BlockSpec((1, _DEC_CB), lambda j: (0, j)),
        ],
        out_specs=pl.BlockSpec((B, _DEC_CB), lambda j: (0, j)),
        out_shape=jax.ShapeDtypeStruct((B, IN_DIM), _f32),
    )(d1, dec_W2, b2_row)


def kernel(x, edge_index, W_msg, b_msg, enc_W1, enc_b1, W_mean, b_mean,
           W_logvar, b_logvar, dec_W1, dec_b1, dec_W2, dec_b2):
    x2d = x.reshape(B * N, F)
    ei_r = edge_index.reshape(B, 2, NS, NCHUNK, CH)

    xbf = x.astype(jnp.bfloat16)
    sums2d, cnt = _sc_agg(xbf, ei_r)
    cnt2d = cnt.reshape(B * N, 1)

    h2d = _conv(sums2d, cnt2d, x2d, W_msg, b_msg.reshape(1, F))
    xs = h2d.reshape(B, IN_DIM)
    eps = jax.random.normal(jax.random.key(42), (B, L), dtype=_f32)
    mean, log_var, d1 = _enc(xs, enc_W1, enc_b1.reshape(1, H),
                             W_mean, b_mean.reshape(1, L),
                             W_logvar, b_logvar.reshape(1, L),
                             dec_W1, dec_b1.reshape(1, H), eps)
    x_hat = _dec(d1, dec_W2, dec_b2.reshape(1, IN_DIM))
    return (x_hat, mean, log_var)
